# Initial kernel scaffold; baseline (speedup 1.0000x reference)
import functools

import jax
import jax.numpy as jnp
from jax import lax
from jax.experimental import pallas as pl
from jax.experimental.pallas import tpu as pltpu
from jax.experimental.pallas import tpu_sc as plsc

NN = 10000
NE = 320000
FD = 64
FP = 80
NEG = 0.2

NC = 2
NS = 16
NW = NC * NS
EPT = NE // NW
CH = 80
NCHUNK = EPT // CH
RPT = NN // NS


def _dense1_body(x_ref, w1_ref, a1s_ref, a1d_ref, r16_ref,
                 h_ref, tabs_ref, tabd_ref, g16_ref, selfmsg_ref):
    x = x_ref[...]
    h = jnp.dot(x, w1_ref[...], preferred_element_type=jnp.float32)
    tabs = jnp.dot(h, a1s_ref[...], preferred_element_type=jnp.float32)
    tabd = jnp.dot(h, a1d_ref[...], preferred_element_type=jnp.float32)
    g = jnp.max(tabs, axis=0, keepdims=True)
    t = tabs + tabd
    num = jnp.maximum(t, NEG * t)
    u = g + tabd
    ub = jnp.maximum(u, NEG * u)
    w = jnp.exp(num - ub)
    wrep = jnp.dot(w, r16_ref[...], preferred_element_type=jnp.float32)
    h_ref[...] = h
    tabs_ref[...] = tabs
    tabd_ref[...] = tabd
    g16_ref[...] = g
    selfmsg_ref[...] = jnp.concatenate([h * wrep, w], axis=1)


def _dense2_body(parts_ref, selfmsg_ref, m80_ref, s80_ref, b1_ref, w2_ref,
                 a2s_ref, a2d_ref, r16_ref,
                 h2_ref, tabs_ref, tabd_ref, g16_ref, selfmsg2_ref):
    acc = parts_ref[0] + parts_ref[1] + selfmsg_ref[...]
    m1 = jnp.dot(acc, m80_ref[...], preferred_element_type=jnp.float32)
    srep = jnp.dot(acc, s80_ref[...], preferred_element_type=jnp.float32)
    h1 = jnp.maximum(m1 / (srep + 1e-16) + b1_ref[...], 0.0)
    h2 = jnp.dot(h1, w2_ref[...], preferred_element_type=jnp.float32)
    tabs = jnp.dot(h2, a2s_ref[...], preferred_element_type=jnp.float32)
    tabd = jnp.dot(h2, a2d_ref[...], preferred_element_type=jnp.float32)
    g = jnp.max(tabs, axis=0, keepdims=True)
    t = tabs + tabd
    num = jnp.maximum(t, NEG * t)
    u = g + tabd
    ub = jnp.maximum(u, NEG * u)
    w = jnp.exp(num - ub)
    wrep = jnp.dot(w, r16_ref[...], preferred_element_type=jnp.float32)
    h2_ref[...] = h2
    tabs_ref[...] = tabs
    tabd_ref[...] = tabd
    g16_ref[...] = g
    selfmsg2_ref[...] = jnp.concatenate([h2 * wrep, w], axis=1)


def _out_body(parts_ref, selfmsg_ref, m80_ref, s80_ref, b2_ref, o_ref):
    acc = parts_ref[0] + parts_ref[1] + selfmsg_ref[...]
    m2 = jnp.dot(acc, m80_ref[...], preferred_element_type=jnp.float32)
    srep = jnp.dot(acc, s80_ref[...], preferred_element_type=jnp.float32)
    o = m2 / (srep + 1e-16) + b2_ref[...]
    col = lax.broadcasted_iota(jnp.int32, (1, FD), 1)
    om = jnp.where(col < 47, o, -1e30)
    mx = jnp.max(om, axis=1, keepdims=True)
    ssum = jnp.sum(jnp.exp(om - mx), axis=1, keepdims=True)
    o_ref[...] = o - (mx + jnp.log(ssum))


def _edge_body(src_hbm, dst_hbm, tabs_hbm, tabd_hbm, h_hbm, g16_hbm, zeros_hbm,
               out_hbm, idx_s, idx_d, sbuf, dbuf, hbuf, msgbuf, g16v,
               acc, sem0, sem1, sem2):
    cid = lax.axis_index("c")
    sid = lax.axis_index("s")
    wid = cid * NS + sid
    pltpu.sync_copy(zeros_hbm, acc.at[pl.ds(sid * RPT, RPT)])
    pltpu.sync_copy(g16_hbm, g16v)
    plsc.subcore_barrier()
    g = g16v[0, :]
    lane = lax.iota(jnp.int32, 16)
    hsel = lane >> 3

    def chunk_body(i, carry):
        base = wid * EPT + i * CH
        pltpu.sync_copy(src_hbm.at[pl.ds(base, CH)], idx_s)
        pltpu.sync_copy(dst_hbm.at[pl.ds(base, CH)], idx_d)
        cs = pltpu.async_copy(tabs_hbm.at[idx_s], sbuf, sem0)
        cd = pltpu.async_copy(tabd_hbm.at[idx_d], dbuf, sem1)
        chh = pltpu.async_copy(h_hbm.at[idx_s], hbuf, sem2)
        cs.wait()
        cd.wait()
        chh.wait()

        def edge_body(k, ecarry):
            s16 = sbuf[k, :]
            d16 = dbuf[k, :]
            t = s16 + d16
            num = jnp.maximum(t, NEG * t)
            u = g + d16
            ub = jnp.maximum(u, NEG * u)
            w = jnp.exp(num - ub)
            msgbuf[k, pl.ds(FD, 16)] = w
            kvec = jnp.full((16,), k, dtype=jnp.int32)
            for sblk in range(4):
                h16 = hbuf[k, pl.ds(sblk * 16, 16)]
                wexp = plsc.load_gather(msgbuf, [kvec, FD + hsel + 2 * sblk])
                msgbuf[k, pl.ds(sblk * 16, 16)] = h16 * wexp
            return ecarry

        lax.fori_loop(0, CH, edge_body, 0)
        pltpu.sync_copy(msgbuf, acc.at[idx_d], add=True)
        return carry

    lax.fori_loop(0, NCHUNK, chunk_body, 0)
    plsc.subcore_barrier()
    pltpu.sync_copy(acc.at[pl.ds(sid * RPT, RPT)],
                    out_hbm.at[cid, pl.ds(sid * RPT, RPT)])


_edge_kernel = functools.partial(
    pl.kernel,
    out_type=jax.ShapeDtypeStruct((NC, NN, FP), jnp.float32),
    mesh=plsc.VectorSubcoreMesh(core_axis_name="c", subcore_axis_name="s"),
    scratch_types=[
        pltpu.VMEM((CH,), jnp.int32),
        pltpu.VMEM((CH,), jnp.int32),
        pltpu.VMEM((CH, 16), jnp.float32),
        pltpu.VMEM((CH, 16), jnp.float32),
        pltpu.VMEM((CH, FD), jnp.float32),
        pltpu.VMEM((CH, FP), jnp.float32),
        pltpu.VMEM((1, 16), jnp.float32),
        pltpu.VMEM_SHARED((NN, FP), jnp.float32),
        pltpu.SemaphoreType.DMA,
        pltpu.SemaphoreType.DMA,
        pltpu.SemaphoreType.DMA,
    ],
)(_edge_body)


def kernel(x, edge_index, W1, att_src1, att_dst1, b1, W2, att_src2, att_dst2, b2):
    f32 = jnp.float32
    src = edge_index[0]
    dst = edge_index[1]
    eye8 = jnp.eye(8, dtype=f32)
    a1s = (eye8[:, None, :] * att_src1[:, :, None]).reshape(FD, 8)
    a1d = (eye8[:, None, :] * att_dst1[:, :, None]).reshape(FD, 8)
    a1s16 = jnp.concatenate([a1s, a1s], axis=1)
    a1d16 = jnp.concatenate([a1d, a1d], axis=1)
    rrep = jnp.kron(eye8, jnp.ones((1, 8), f32))
    r16 = jnp.concatenate([rrep, jnp.zeros((8, FD), f32)], axis=0)
    m80 = jnp.concatenate([jnp.eye(FD, dtype=f32), jnp.zeros((16, FD), f32)], axis=0)
    s80 = jnp.concatenate([jnp.zeros((FD, FD), f32), rrep,
                           jnp.zeros((8, FD), f32)], axis=0)
    w2p = jnp.zeros((FD, FD), f32).at[:, :47].set(W2)
    a2s = jnp.zeros((FD,), f32).at[:47].set(att_src2[0])
    a2d = jnp.zeros((FD,), f32).at[:47].set(att_dst2[0])
    a2s16 = jnp.broadcast_to(a2s[:, None], (FD, 16))
    a2d16 = jnp.broadcast_to(a2d[:, None], (FD, 16))
    b1r = b1.reshape(1, FD)
    b2p = jnp.zeros((1, FD), f32).at[0, :47].set(b2)
    zeros_blk = jnp.zeros((RPT, FP), f32)

    h1, tabs1, tabd1, g16a, selfmsg1 = pl.pallas_call(
        _dense1_body,
        out_shape=[
            jax.ShapeDtypeStruct((NN, FD), f32),
            jax.ShapeDtypeStruct((NN, 16), f32),
            jax.ShapeDtypeStruct((NN, 16), f32),
            jax.ShapeDtypeStruct((1, 16), f32),
            jax.ShapeDtypeStruct((NN, FP), f32),
        ],
    )(x, W1, a1s16, a1d16, r16)

    parts1 = _edge_kernel(src, dst, tabs1, tabd1, h1, g16a, zeros_blk)

    h2, tabs2, tabd2, g16b, selfmsg2 = pl.pallas_call(
        _dense2_body,
        out_shape=[
            jax.ShapeDtypeStruct((NN, FD), f32),
            jax.ShapeDtypeStruct((NN, 16), f32),
            jax.ShapeDtypeStruct((NN, 16), f32),
            jax.ShapeDtypeStruct((1, 16), f32),
            jax.ShapeDtypeStruct((NN, FP), f32),
        ],
    )(parts1, selfmsg1, m80, s80, b1r, w2p, a2s16, a2d16, r16)

    parts2 = _edge_kernel(src, dst, tabs2, tabd2, h2, g16b, zeros_blk)

    out = pl.pallas_call(
        _out_body,
        out_shape=jax.ShapeDtypeStruct((NN, FD), f32),
    )(parts2, selfmsg2, m80, s80, b2p)
    return out[:, :47]


# speedup vs baseline: 35.8586x; 35.8586x over previous
import functools

import jax
import jax.numpy as jnp
from jax import lax
from jax.experimental import pallas as pl
from jax.experimental.pallas import tpu as pltpu
from jax.experimental.pallas import tpu_sc as plsc

NN = 10000
NE = 320000
FD = 64
FR = 128
NEG = 0.2

NC = 2
NS = 16
NW = NC * NS
EPT = NE // NW
CH = 80
NCHUNK = EPT // CH
NP = 10240
RPT = NP // NS


def _attn_tail(h, tabs, tabd, r16):
    g = jnp.max(tabs, axis=0, keepdims=True)
    t = tabs + tabd
    num = jnp.maximum(t, NEG * t)
    u = g + tabd
    ub = jnp.maximum(u, NEG * u)
    w = jnp.exp(num - ub)
    wrep = jnp.dot(w, r16, preferred_element_type=jnp.float32)
    zpad = jnp.zeros((h.shape[0], FR - FD - 32), jnp.float32)
    table = jnp.concatenate([h, tabs, tabd, zpad], axis=1)
    selfmsg = jnp.concatenate([h * wrep, w, jnp.zeros_like(tabs),
                               zpad], axis=1)
    return g, table, selfmsg


def _dense1_body(x_ref, w1_ref, a1s_ref, a1d_ref, r16_ref,
                 tab_ref, g16_ref, selfmsg_ref):
    x = x_ref[...]
    h = jnp.dot(x, w1_ref[...], preferred_element_type=jnp.float32)
    tabs = jnp.dot(h, a1s_ref[...], preferred_element_type=jnp.float32)
    tabd = jnp.dot(h, a1d_ref[...], preferred_element_type=jnp.float32)
    g, table, selfmsg = _attn_tail(h, tabs, tabd, r16_ref[...])
    tab_ref[...] = table
    g16_ref[...] = g
    selfmsg_ref[...] = selfmsg


def _dense2_body(parts_ref, selfmsg_ref, m128_ref, s128_ref, b1_ref, w2_ref,
                 a2s_ref, a2d_ref, r16_ref,
                 tab_ref, g16_ref, selfmsg2_ref):
    acc = parts_ref[0, :NN] + parts_ref[1, :NN] + selfmsg_ref[...]
    m1 = jnp.dot(acc, m128_ref[...], preferred_element_type=jnp.float32)
    srep = jnp.dot(acc, s128_ref[...], preferred_element_type=jnp.float32)
    h1 = jnp.maximum(m1 / (srep + 1e-16) + b1_ref[...], 0.0)
    h2 = jnp.dot(h1, w2_ref[...], preferred_element_type=jnp.float32)
    tabs = jnp.dot(h2, a2s_ref[...], preferred_element_type=jnp.float32)
    tabd = jnp.dot(h2, a2d_ref[...], preferred_element_type=jnp.float32)
    g, table, selfmsg = _attn_tail(h2, tabs, tabd, r16_ref[...])
    tab_ref[...] = table
    g16_ref[...] = g
    selfmsg2_ref[...] = selfmsg


def _out_body(parts_ref, selfmsg_ref, m128_ref, s128_ref, b2_ref, o_ref):
    acc = parts_ref[0, :NN] + parts_ref[1, :NN] + selfmsg_ref[...]
    m2 = jnp.dot(acc, m128_ref[...], preferred_element_type=jnp.float32)
    srep = jnp.dot(acc, s128_ref[...], preferred_element_type=jnp.float32)
    o = m2 / (srep + 1e-16) + b2_ref[...]
    col = lax.broadcasted_iota(jnp.int32, (1, FD), 1)
    om = jnp.where(col < 47, o, -1e30)
    mx = jnp.max(om, axis=1, keepdims=True)
    ssum = jnp.sum(jnp.exp(om - mx), axis=1, keepdims=True)
    o_ref[...] = o - (mx + jnp.log(ssum))


def _edge_body(src_hbm, dst_hbm, tab_hbm, g16_hbm, zeros_hbm,
               out_hbm, idx_s, idx_d, srcbuf, dstbuf, msgbuf, wbuf, g16v,
               acc, sem0, sem1):
    cid = lax.axis_index("c")
    sid = lax.axis_index("s")
    wid = cid * NS + sid
    pltpu.sync_copy(zeros_hbm, acc.at[pl.ds(sid * RPT, RPT)])
    pltpu.sync_copy(zeros_hbm.at[pl.ds(0, CH)], msgbuf)
    pltpu.sync_copy(g16_hbm, g16v)
    plsc.subcore_barrier()
    g = g16v[0, :]
    lane = lax.iota(jnp.int32, 16)
    hsel = lane >> 3

    def chunk_body(i, carry):
        base = wid * EPT + i * CH
        pltpu.sync_copy(src_hbm.at[pl.ds(base, CH)], idx_s)
        pltpu.sync_copy(dst_hbm.at[pl.ds(base, CH)], idx_d)
        cs = pltpu.async_copy(tab_hbm.at[idx_s], srcbuf, sem0)
        cd = pltpu.async_copy(tab_hbm.at[idx_d], dstbuf, sem1)
        cs.wait()
        cd.wait()

        def edge_body(k, ecarry):
            s16 = srcbuf[k, pl.ds(FD, 16)]
            d16 = dstbuf[k, pl.ds(FD + 16, 16)]
            t = s16 + d16
            num = jnp.maximum(t, NEG * t)
            u = g + d16
            ub = jnp.maximum(u, NEG * u)
            w = jnp.exp(num - ub)
            msgbuf[k, pl.ds(FD, 16)] = w
            wbuf[pl.ds(k * 16, 16)] = w
            for sblk in range(4):
                h16 = srcbuf[k, pl.ds(sblk * 16, 16)]
                wexp = plsc.load_gather(wbuf, [k * 16 + 2 * sblk + hsel])
                msgbuf[k, pl.ds(sblk * 16, 16)] = h16 * wexp
            return ecarry

        lax.fori_loop(0, CH, edge_body, 0)
        pltpu.sync_copy(msgbuf, acc.at[idx_d], add=True)
        return carry

    lax.fori_loop(0, NCHUNK, chunk_body, 0)
    plsc.subcore_barrier()
    pltpu.sync_copy(acc.at[pl.ds(sid * RPT, RPT)],
                    out_hbm.at[cid, pl.ds(sid * RPT, RPT)])


_edge_kernel = functools.partial(
    pl.kernel,
    out_type=jax.ShapeDtypeStruct((NC, NP, FR), jnp.float32),
    mesh=plsc.VectorSubcoreMesh(core_axis_name="c", subcore_axis_name="s"),
    compiler_params=pltpu.CompilerParams(needs_layout_passes=False),
    scratch_types=[
        pltpu.VMEM((CH,), jnp.int32),
        pltpu.VMEM((CH,), jnp.int32),
        pltpu.VMEM((CH, FR), jnp.float32),
        pltpu.VMEM((CH, FR), jnp.float32),
        pltpu.VMEM((CH, FR), jnp.float32),
        pltpu.VMEM((CH * 16,), jnp.float32),
        pltpu.VMEM((1, 16), jnp.float32),
        pltpu.VMEM_SHARED((NP, FR), jnp.float32),
        pltpu.SemaphoreType.DMA,
        pltpu.SemaphoreType.DMA,
    ],
)(_edge_body)


def kernel(x, edge_index, W1, att_src1, att_dst1, b1, W2, att_src2, att_dst2, b2):
    f32 = jnp.float32
    src = edge_index[0]
    dst = edge_index[1]
    eye8 = jnp.eye(8, dtype=f32)
    a1s = (eye8[:, None, :] * att_src1[:, :, None]).reshape(FD, 8)
    a1d = (eye8[:, None, :] * att_dst1[:, :, None]).reshape(FD, 8)
    a1s16 = jnp.concatenate([a1s, a1s], axis=1)
    a1d16 = jnp.concatenate([a1d, a1d], axis=1)
    rrep = jnp.kron(eye8, jnp.ones((1, 8), f32))
    r16 = jnp.concatenate([rrep, jnp.zeros((8, FD), f32)], axis=0)
    m128 = jnp.concatenate([jnp.eye(FD, dtype=f32),
                            jnp.zeros((FR - FD, FD), f32)], axis=0)
    s128 = jnp.concatenate([jnp.zeros((FD, FD), f32), rrep,
                            jnp.zeros((FR - FD - 8, FD), f32)], axis=0)
    w2p = jnp.zeros((FD, FD), f32).at[:, :47].set(W2)
    a2s = jnp.zeros((FD,), f32).at[:47].set(att_src2[0])
    a2d = jnp.zeros((FD,), f32).at[:47].set(att_dst2[0])
    a2s16 = jnp.broadcast_to(a2s[:, None], (FD, 16))
    a2d16 = jnp.broadcast_to(a2d[:, None], (FD, 16))
    b1r = b1.reshape(1, FD)
    b2p = jnp.zeros((1, FD), f32).at[0, :47].set(b2)
    zeros_blk = jnp.zeros((RPT, FR), f32)

    tab1, g16a, selfmsg1 = pl.pallas_call(
        _dense1_body,
        out_shape=[
            jax.ShapeDtypeStruct((NN, FR), f32),
            jax.ShapeDtypeStruct((1, 16), f32),
            jax.ShapeDtypeStruct((NN, FR), f32),
        ],
    )(x, W1, a1s16, a1d16, r16)

    parts1 = _edge_kernel(src, dst, tab1, g16a, zeros_blk)

    tab2, g16b, selfmsg2 = pl.pallas_call(
        _dense2_body,
        out_shape=[
            jax.ShapeDtypeStruct((NN, FR), f32),
            jax.ShapeDtypeStruct((1, 16), f32),
            jax.ShapeDtypeStruct((NN, FR), f32),
        ],
    )(parts1, selfmsg1, m128, s128, b1r, w2p, a2s16, a2d16, r16)

    parts2 = _edge_kernel(src, dst, tab2, g16b, zeros_blk)

    out = pl.pallas_call(
        _out_body,
        out_shape=jax.ShapeDtypeStruct((NN, FD), f32),
    )(parts2, selfmsg2, m128, s128, b2p)
    return out[:, :47]


# speedup vs baseline: 88.3363x; 2.4635x over previous
import functools

import jax
import jax.numpy as jnp
from jax import lax
from jax.experimental import pallas as pl
from jax.experimental.pallas import tpu as pltpu
from jax.experimental.pallas import tpu_sc as plsc

NN = 10000
NE = 320000
FD = 64
FR = 128
NEG = 0.2

NC = 2
NS = 16
NW = NC * NS
EPT = NE // NW
CH = 40
NCHUNK = EPT // CH
NP = 10240
RPT = NP // NS


def _mk_tables(h, asrce, adste):
    ge = jnp.max(asrce, axis=0, keepdims=True)
    u = ge + adste
    ube = jnp.maximum(u, NEG * u)
    t = asrce + adste
    ws = jnp.exp(jnp.maximum(t, NEG * t) - ube)
    taba = jnp.concatenate([h, asrce], axis=1)
    tabb = jnp.concatenate([adste, ube], axis=1)
    selfmsg = jnp.concatenate([h * ws, ws], axis=1)
    return taba, tabb, selfmsg


def _dense1_body(x_ref, w1_ref, a1s_ref, a1d_ref, rrep_ref,
                 taba_ref, tabb_ref, selfmsg_ref):
    x = x_ref[...]
    h = jnp.dot(x, w1_ref[...], preferred_element_type=jnp.float32)
    a_src = jnp.dot(h, a1s_ref[...], preferred_element_type=jnp.float32)
    a_dst = jnp.dot(h, a1d_ref[...], preferred_element_type=jnp.float32)
    asrce = jnp.dot(a_src, rrep_ref[...], preferred_element_type=jnp.float32)
    adste = jnp.dot(a_dst, rrep_ref[...], preferred_element_type=jnp.float32)
    taba, tabb, selfmsg = _mk_tables(h, asrce, adste)
    taba_ref[...] = taba
    tabb_ref[...] = tabb
    selfmsg_ref[...] = selfmsg


def _dense2_body(parts_ref, selfmsg_ref, m128_ref, s128_ref, b1_ref, w2_ref,
                 a2s_ref, a2d_ref,
                 taba_ref, tabb_ref, selfmsg2_ref):
    acc = parts_ref[0, :NN] + parts_ref[1, :NN] + selfmsg_ref[...]
    m1 = jnp.dot(acc, m128_ref[...], preferred_element_type=jnp.float32)
    srep = jnp.dot(acc, s128_ref[...], preferred_element_type=jnp.float32)
    h1 = jnp.maximum(m1 / (srep + 1e-16) + b1_ref[...], 0.0)
    h2 = jnp.dot(h1, w2_ref[...], preferred_element_type=jnp.float32)
    asrce = jnp.dot(h2, a2s_ref[...], preferred_element_type=jnp.float32)
    adste = jnp.dot(h2, a2d_ref[...], preferred_element_type=jnp.float32)
    taba, tabb, selfmsg = _mk_tables(h2, asrce, adste)
    taba_ref[...] = taba
    tabb_ref[...] = tabb
    selfmsg2_ref[...] = selfmsg


def _out_body(parts_ref, selfmsg_ref, m128_ref, s128_ref, b2_ref, o_ref):
    acc = parts_ref[0, :NN] + parts_ref[1, :NN] + selfmsg_ref[...]
    m2 = jnp.dot(acc, m128_ref[...], preferred_element_type=jnp.float32)
    srep = jnp.dot(acc, s128_ref[...], preferred_element_type=jnp.float32)
    o = m2 / (srep + 1e-16) + b2_ref[...]
    col = lax.broadcasted_iota(jnp.int32, (1, FD), 1)
    om = jnp.where(col < 47, o, -1e30)
    mx = jnp.max(om, axis=1, keepdims=True)
    ssum = jnp.sum(jnp.exp(om - mx), axis=1, keepdims=True)
    o_ref[...] = o - (mx + jnp.log(ssum))


def _edge_body(src_hbm, dst_hbm, taba_hbm, tabb_hbm, zeros_hbm, out_hbm,
               idxd, is0, is1, sb0, db0, mb0, si0, sb1, db1, mb1, si1,
               acc, gs0, gd0, ss0, is0sem, gs1, gd1, ss1, is1sem):
    cid = lax.axis_index("c")
    sid = lax.axis_index("s")
    wid = cid * NS + sid
    pltpu.sync_copy(zeros_hbm, acc.at[pl.ds(sid * RPT, RPT)])
    ebase = wid * EPT
    pltpu.sync_copy(dst_hbm.at[pl.ds(ebase, EPT)], idxd)
    plsc.subcore_barrier()

    def fire_sidx(c, isb, isem):
        pltpu.async_copy(src_hbm.at[pl.ds(ebase + c * CH, CH)], isb, isem)

    def wait_sidx(c, isb, isem):
        pltpu.make_async_copy(
            src_hbm.at[pl.ds(ebase + c * CH, CH)], isb, isem).wait()

    def fire_gather(c, isb, sb, db, gs, gd):
        pltpu.async_copy(taba_hbm.at[isb], sb, gs)
        pltpu.async_copy(tabb_hbm.at[idxd.at[pl.ds(c * CH, CH)]], db, gd)

    def wait_gather(c, isb, sb, db, gs, gd):
        pltpu.make_async_copy(taba_hbm.at[isb], sb, gs).wait()
        pltpu.make_async_copy(
            tabb_hbm.at[idxd.at[pl.ds(c * CH, CH)]], db, gd).wait()

    def copy_scat_idx(c, si):
        si[pl.ds(0, 16)] = idxd[pl.ds(c * CH, 16)]
        si[pl.ds(16, 16)] = idxd[pl.ds(c * CH + 16, 16)]
        si[pl.ds(24, 16)] = idxd[pl.ds(c * CH + 24, 16)]

    def compute(sb, db, mb):
        @plsc.parallel_loop(0, CH, step=1, unroll=4)
        def edge(k):
            for s in range(4):
                o = 16 * s
                h16 = sb[k, pl.ds(o, 16)]
                se = sb[k, pl.ds(FD + o, 16)]
                de = db[k, pl.ds(o, 16)]
                ue = db[k, pl.ds(FD + o, 16)]
                t = se + de
                w = jnp.exp(jnp.maximum(t, NEG * t) - ue)
                mb[k, pl.ds(FD + o, 16)] = w
                mb[k, pl.ds(o, 16)] = h16 * w

    def fire_scatter(mb, si, ss):
        pltpu.async_copy(mb, acc.at[si], ss, add=True)

    def wait_scatter(mb, si, ss):
        pltpu.make_async_copy(mb, acc.at[si], ss).wait()

    pltpu.sync_copy(src_hbm.at[pl.ds(ebase, CH)], is0)
    fire_gather(0, is0, sb0, db0, gs0, gd0)
    fire_sidx(1, is1, is1sem)

    def pair(j, carry):
        a = 2 * j
        wait_gather(a, is0, sb0, db0, gs0, gd0)
        wait_sidx(a + 1, is1, is1sem)
        fire_gather(a + 1, is1, sb1, db1, gs1, gd1)

        @pl.when(a + 2 < NCHUNK)
        def _():
            fire_sidx(a + 2, is0, is0sem)

        @pl.when(j > 0)
        def _():
            wait_scatter(mb0, si0, ss0)

        copy_scat_idx(a, si0)
        compute(sb0, db0, mb0)
        fire_scatter(mb0, si0, ss0)

        wait_gather(a + 1, is1, sb1, db1, gs1, gd1)

        @pl.when(a + 2 < NCHUNK)
        def _():
            wait_sidx(a + 2, is0, is0sem)
            fire_gather(a + 2, is0, sb0, db0, gs0, gd0)
            fire_sidx(a + 3, is1, is1sem)

        @pl.when(j > 0)
        def _():
            wait_scatter(mb1, si1, ss1)

        copy_scat_idx(a + 1, si1)
        compute(sb1, db1, mb1)
        fire_scatter(mb1, si1, ss1)
        return carry

    lax.fori_loop(0, NCHUNK // 2, pair, 0)

    wait_scatter(mb0, si0, ss0)
    wait_scatter(mb1, si1, ss1)

    plsc.subcore_barrier()
    pltpu.sync_copy(acc.at[pl.ds(sid * RPT, RPT)],
                    out_hbm.at[cid, pl.ds(sid * RPT, RPT)])


_edge_kernel = functools.partial(
    pl.kernel,
    out_type=jax.ShapeDtypeStruct((NC, NP, FR), jnp.float32),
    mesh=plsc.VectorSubcoreMesh(core_axis_name="c", subcore_axis_name="s"),
    compiler_params=pltpu.CompilerParams(needs_layout_passes=False),
    scratch_types=[
        pltpu.VMEM((EPT,), jnp.int32),
        pltpu.VMEM((CH,), jnp.int32),
        pltpu.VMEM((CH,), jnp.int32),
        pltpu.VMEM((CH, FR), jnp.float32),
        pltpu.VMEM((CH, FR), jnp.float32),
        pltpu.VMEM((CH, FR), jnp.float32),
        pltpu.VMEM((CH,), jnp.int32),
        pltpu.VMEM((CH, FR), jnp.float32),
        pltpu.VMEM((CH, FR), jnp.float32),
        pltpu.VMEM((CH, FR), jnp.float32),
        pltpu.VMEM((CH,), jnp.int32),
        pltpu.VMEM_SHARED((NP, FR), jnp.float32),
        pltpu.SemaphoreType.DMA,
        pltpu.SemaphoreType.DMA,
        pltpu.SemaphoreType.DMA,
        pltpu.SemaphoreType.DMA,
        pltpu.SemaphoreType.DMA,
        pltpu.SemaphoreType.DMA,
        pltpu.SemaphoreType.DMA,
        pltpu.SemaphoreType.DMA,
    ],
)(_edge_body)


def kernel(x, edge_index, W1, att_src1, att_dst1, b1, W2, att_src2, att_dst2, b2):
    f32 = jnp.float32
    src = edge_index[0]
    dst = edge_index[1]
    eye8 = jnp.eye(8, dtype=f32)
    a1s = (eye8[:, None, :] * att_src1[:, :, None]).reshape(FD, 8)
    a1d = (eye8[:, None, :] * att_dst1[:, :, None]).reshape(FD, 8)
    rrep = jnp.kron(eye8, jnp.ones((1, 8), f32))
    m128 = jnp.concatenate([jnp.eye(FD, dtype=f32),
                            jnp.zeros((FD, FD), f32)], axis=0)
    s128 = jnp.concatenate([jnp.zeros((FD, FD), f32),
                            jnp.eye(FD, dtype=f32)], axis=0)
    w2p = jnp.zeros((FD, FD), f32).at[:, :47].set(W2)
    a2s = jnp.zeros((FD,), f32).at[:47].set(att_src2[0])
    a2d = jnp.zeros((FD,), f32).at[:47].set(att_dst2[0])
    a2se = jnp.broadcast_to(a2s[:, None], (FD, FD))
    a2de = jnp.broadcast_to(a2d[:, None], (FD, FD))
    b1r = b1.reshape(1, FD)
    b2p = jnp.zeros((1, FD), f32).at[0, :47].set(b2)
    zeros_blk = jnp.zeros((RPT, FR), f32)

    taba1, tabb1, selfmsg1 = pl.pallas_call(
        _dense1_body,
        out_shape=[
            jax.ShapeDtypeStruct((NN, FR), f32),
            jax.ShapeDtypeStruct((NN, FR), f32),
            jax.ShapeDtypeStruct((NN, FR), f32),
        ],
    )(x, W1, a1s, a1d, rrep)

    parts1 = _edge_kernel(src, dst, taba1, tabb1, zeros_blk)

    taba2, tabb2, selfmsg2 = pl.pallas_call(
        _dense2_body,
        out_shape=[
            jax.ShapeDtypeStruct((NN, FR), f32),
            jax.ShapeDtypeStruct((NN, FR), f32),
            jax.ShapeDtypeStruct((NN, FR), f32),
        ],
    )(parts1, selfmsg1, m128, s128, b1r, w2p, a2se, a2de)

    parts2 = _edge_kernel(src, dst, taba2, tabb2, zeros_blk)

    out = pl.pallas_call(
        _out_body,
        out_shape=jax.ShapeDtypeStruct((NN, FD), f32),
    )(parts2, selfmsg2, m128, s128, b2p)
    return out[:, :47]
